# Initial kernel scaffold; baseline (speedup 1.0000x reference)
import jax
import jax.numpy as jnp
from jax.experimental import pallas as pl


def kernel(z, codebook, noise):
    raise NotImplementedError("write your pallas kernel here")


# speedup vs baseline: 1.1212x; 1.1212x over previous
import jax
import jax.numpy as jnp
from jax.experimental import pallas as pl
from jax.experimental.pallas import tpu as pltpu

N = 16384
D = 32
K = 8192
BN = 512
BK = 4096


def _argmin_body(z_ref, zsq_ref, ct_ref, out_ref):
    zb = z_ref[...]
    zsq = zsq_ref[...]
    bestv = jnp.full((BN, 1), jnp.inf, jnp.float32)
    besti = jnp.zeros((BN, 1), jnp.int32)
    for j in range(K // BK):
        ct = ct_ref[:, j * BK:(j + 1) * BK]
        csq = jnp.sum(ct * ct, axis=0, keepdims=True)
        p = jnp.dot(zb.astype(jnp.bfloat16), ct.astype(jnp.bfloat16),
                    preferred_element_type=jnp.float32)
        d = (zsq + csq) - 2.0 * p
        cmin = jnp.min(d, axis=1, keepdims=True)
        ids = jax.lax.broadcasted_iota(jnp.int32, (BN, BK), 1)
        cidx = jnp.min(jnp.where(d == cmin, ids, BK), axis=1, keepdims=True)
        cidx = cidx + j * BK
        take = cmin < bestv
        bestv = jnp.where(take, cmin, bestv)
        besti = jnp.where(take, cidx, besti)
        bestv = bestv.astype(jnp.bfloat16).astype(jnp.float32)
    out_ref[...] = besti


def _argmin_indices(z, zsq, codebook_t, interpret=False):
    out = pl.pallas_call(
        _argmin_body,
        grid=(N // BN,),
        in_specs=[
            pl.BlockSpec((BN, D), lambda i: (i, 0)),
            pl.BlockSpec((BN, 1), lambda i: (i, 0)),
            pl.BlockSpec((D, K), lambda i: (0, 0)),
        ],
        out_specs=pl.BlockSpec((BN, 1), lambda i: (i, 0)),
        out_shape=jax.ShapeDtypeStruct((N, 1), jnp.int32),
        interpret=interpret,
    )(z, zsq, codebook_t)
    return out[:, 0]


def kernel(z, codebook, noise):
    zsq = jnp.sum(z ** 2, axis=1)[:, None]
    indices = _argmin_indices(z, zsq, codebook.T)
    z_hard = jnp.take(codebook, indices, axis=0)
    direction = z_hard - z
    random_vectors = noise + direction
    nrm = jnp.maximum(jnp.linalg.norm(random_vectors, axis=1, keepdims=True), 1e-12)
    normalized = random_vectors / nrm
    error_magnitude = jnp.linalg.norm(z_hard - z, axis=1, keepdims=True)
    vq_error = error_magnitude * jax.lax.stop_gradient(normalized)
    z_q = z + vq_error
    encodings = jax.nn.one_hot(indices, K, dtype=jnp.float32)
    avg_probs = jnp.mean(encodings, axis=0)
    perplexity = jnp.exp(-jnp.sum(avg_probs * jnp.log(avg_probs + 1e-10)))
    return (z_q, indices, perplexity)


# speedup vs baseline: 1.6408x; 1.4635x over previous
import jax
import jax.numpy as jnp
from jax import lax
from jax.experimental import pallas as pl
from jax.experimental.pallas import tpu as pltpu
from jax.experimental.pallas import tpu_sc as plsc

N = 16384
D = 32
K = 8192
BN = 512
BK = 4096

NC = 2
NS = 16
NW = NC * NS
RPW = N // NW
RCH = 128
NCH = RPW // RCH


def _argmin_body(z_ref, zsq_ref, ct_ref, out_ref):
    zb = z_ref[...]
    zsq = zsq_ref[...]
    bestv = jnp.full((BN, 1), jnp.inf, jnp.float32)
    besti = jnp.zeros((BN, 1), jnp.int32)
    for j in range(K // BK):
        ct = ct_ref[:, j * BK:(j + 1) * BK]
        csq = jnp.sum(ct * ct, axis=0, keepdims=True)
        p = jnp.dot(zb.astype(jnp.bfloat16), ct.astype(jnp.bfloat16),
                    preferred_element_type=jnp.float32)
        d = (zsq + csq) - 2.0 * p
        cmin = jnp.min(d, axis=1, keepdims=True)
        ids = jax.lax.broadcasted_iota(jnp.int32, (BN, BK), 1)
        cidx = jnp.min(jnp.where(d == cmin, ids, BK), axis=1, keepdims=True)
        cidx = cidx + j * BK
        take = cmin < bestv
        bestv = jnp.where(take, cmin, bestv)
        besti = jnp.where(take, cidx, besti)
        bestv = bestv.astype(jnp.bfloat16).astype(jnp.float32)
    out_ref[...] = besti


def _argmin_indices(z, zsq, codebook_t):
    out = pl.pallas_call(
        _argmin_body,
        grid=(N // BN,),
        in_specs=[
            pl.BlockSpec((BN, D), lambda i: (i, 0)),
            pl.BlockSpec((BN, 1), lambda i: (i, 0)),
            pl.BlockSpec((D, K), lambda i: (0, 0)),
        ],
        out_specs=pl.BlockSpec((BN, 1), lambda i: (i, 0)),
        out_shape=jax.ShapeDtypeStruct((N, 1), jnp.int32),
    )(z, zsq, codebook_t)
    return out[:, 0]


def _sc_body(cb_ref, idx_hbm, zhard_ref, hist_ref,
             idx_v, rows_v, stage_v, ones_v, hist_sh, sem):
    c = lax.axis_index("c")
    s = lax.axis_index("s")
    wid = s * NC + c
    base = wid * RPW

    for j in range(NCH):
        pltpu.sync_copy(idx_hbm.at[pl.ds(base + j * RCH, RCH)], idx_v.at[j])

    descs = [pltpu.async_copy(cb_ref.at[idx_v.at[j]], rows_v.at[j], sem)
             for j in range(NCH)]
    for dsc in descs:
        dsc.wait()
    for j in range(NCH):
        pltpu.sync_copy(rows_v.at[j], zhard_ref.at[pl.ds(base + j * RCH, RCH)])

    def _zero(i, _):
        stage_v[pl.ds(i * 16, 16)] = jnp.zeros((16,), jnp.float32)
        return 0
    lax.fori_loop(0, RPW // 16, _zero, 0)
    pltpu.sync_copy(stage_v, hist_sh.at[pl.ds(s * RPW, RPW)])

    def _ones(i, _):
        ones_v[pl.ds(i * 16, 16)] = jnp.ones((16,), jnp.float32)
        return 0
    lax.fori_loop(0, RCH // 16, _ones, 0)

    plsc.subcore_barrier()

    for j in range(NCH):
        pltpu.sync_copy(ones_v, hist_sh.at[idx_v.at[j]], add=True)

    plsc.subcore_barrier()

    pltpu.sync_copy(hist_sh.at[pl.ds(s * RPW, RPW)], stage_v)
    pltpu.sync_copy(stage_v, hist_ref.at[c, pl.ds(s * RPW, RPW)])


def _sc_gather_hist(codebook, indices):
    mesh = plsc.VectorSubcoreMesh(core_axis_name="c", subcore_axis_name="s")
    f = pl.kernel(
        _sc_body,
        out_type=[
            jax.ShapeDtypeStruct((N, D), jnp.float32),
            jax.ShapeDtypeStruct((NC, K), jnp.float32),
        ],
        mesh=mesh,
        scratch_types=[
            pltpu.VMEM((NCH, RCH), jnp.int32),
            pltpu.VMEM((NCH, RCH, D), jnp.float32),
            pltpu.VMEM((RPW,), jnp.float32),
            pltpu.VMEM((RCH,), jnp.float32),
            pltpu.VMEM_SHARED((K,), jnp.float32),
            pltpu.SemaphoreType.DMA,
        ],
        compiler_params=pltpu.CompilerParams(use_tc_tiling_on_sc=False),
    )
    return f(codebook, indices)


def _tail_body(z_ref, zh_ref, noise_ref, hist_ref, zq_ref, perp_ref):
    zb = z_ref[...]
    zh = zh_ref[...]
    nb = noise_ref[...]
    direction = zh - zb
    rv = nb + direction
    nrm = jnp.maximum(jnp.sqrt(jnp.sum(rv * rv, axis=1, keepdims=True)), 1e-12)
    err = jnp.sqrt(jnp.sum(direction * direction, axis=1, keepdims=True))
    zq_ref[...] = zb + err * (rv / nrm)

    counts = hist_ref[0:1, :] + hist_ref[1:2, :]
    probs = counts * (1.0 / N)
    ent = jnp.sum(probs * jnp.log(probs + 1e-10), keepdims=True)
    perp_ref[...] = jnp.exp(-ent)


def _tail(z, z_hard, noise, hist):
    zq, perp = pl.pallas_call(
        _tail_body,
        out_shape=[
            jax.ShapeDtypeStruct((N, D), jnp.float32),
            jax.ShapeDtypeStruct((1, 1), jnp.float32),
        ],
    )(z, z_hard, noise, hist)
    return zq, perp[0, 0]


def kernel(z, codebook, noise):
    zsq = jnp.sum(z ** 2, axis=1)[:, None]
    indices = _argmin_indices(z, zsq, codebook.T)
    z_hard, hist = _sc_gather_hist(codebook, indices)
    z_q, perplexity = _tail(z, z_hard, noise, hist)
    return (z_q, indices, perplexity)


# speedup vs baseline: 1.9888x; 1.2121x over previous
import jax
import jax.numpy as jnp
from jax import lax
from jax.experimental import pallas as pl
from jax.experimental.pallas import tpu as pltpu
from jax.experimental.pallas import tpu_sc as plsc

N = 16384
D = 32
K = 8192
BN = 512
BK = 4096

NC = 2
NS = 16
NW = NC * NS
RPW = N // NW
RCH = 128
NCH = RPW // RCH


RB = 64
LG = 128


def _argmin_body(z_ref, zsq_ref, ct2_ref, out_ref):
    zb = z_ref[...]
    ct2 = ct2_ref[...]
    csq = jnp.sum(ct2 * ct2, axis=0, keepdims=True) * 0.25
    zb16 = zb.astype(jnp.bfloat16)
    p2 = [jnp.dot(zb16, ct2[:, j * BK:(j + 1) * BK].astype(jnp.bfloat16),
                  preferred_element_type=jnp.float32)
          for j in range(K // BK)]
    zsq = zsq_ref[...]

    results = []
    for rb in range(BN // RB):
        rows = slice(rb * RB, (rb + 1) * RB)
        zsq_rb = zsq[rows, :]
        lane = jax.lax.broadcasted_iota(jnp.int32, (RB, LG), 1)
        bestv = jnp.full((RB, 1), jnp.inf, jnp.float32)
        besti = jnp.zeros((RB, 1), jnp.int32)
        for j in range(K // BK):
            acc_v = None
            for g in range(BK // LG):
                cols = slice(g * LG, (g + 1) * LG)
                u = zsq_rb + csq[:, j * BK + g * LG: j * BK + (g + 1) * LG]
                dg = u - p2[j][rows, cols]
                if acc_v is None:
                    acc_v = dg
                    acc_c = jnp.zeros((RB, LG), jnp.int32)
                else:
                    ch = dg < acc_v
                    acc_v = jnp.where(ch, dg, acc_v)
                    acc_c = jnp.where(ch, jnp.full((RB, LG), g, jnp.int32), acc_c)
            cmin = jnp.min(acc_v, axis=1, keepdims=True)
            kk = acc_c * LG + lane
            cand = jnp.min(jnp.where(acc_v == cmin, kk, BK),
                           axis=1, keepdims=True) + j * BK
            take = cmin < bestv
            bestv = jnp.where(take, cmin, bestv)
            besti = jnp.where(take, cand, besti)
            bestv = bestv.astype(jnp.bfloat16).astype(jnp.float32)
        results.append(besti)
    out_ref[...] = jnp.concatenate(results, axis=0)


def _argmin_indices(z, zsq, codebook_t2):
    out = pl.pallas_call(
        _argmin_body,
        grid=(N // BN,),
        in_specs=[
            pl.BlockSpec((BN, D), lambda i: (i, 0)),
            pl.BlockSpec((BN, 1), lambda i: (i, 0)),
            pl.BlockSpec((D, K), lambda i: (0, 0)),
        ],
        out_specs=pl.BlockSpec((BN, 1), lambda i: (i, 0)),
        out_shape=jax.ShapeDtypeStruct((N, 1), jnp.int32),
    )(z, zsq, codebook_t2)
    return out[:, 0]


def _sc_body(cb_ref, idx_hbm, zhard_ref, hist_ref,
             idx_v, rows_v, stage_v, ones_v, hist_sh, sem):
    c = lax.axis_index("c")
    s = lax.axis_index("s")
    wid = s * NC + c
    base = wid * RPW

    for j in range(NCH):
        pltpu.sync_copy(idx_hbm.at[pl.ds(base + j * RCH, RCH)], idx_v.at[j])

    descs = [pltpu.async_copy(cb_ref.at[idx_v.at[j]], rows_v.at[j], sem)
             for j in range(NCH)]
    for dsc in descs:
        dsc.wait()
    for j in range(NCH):
        pltpu.sync_copy(rows_v.at[j], zhard_ref.at[pl.ds(base + j * RCH, RCH)])

    def _zero(i, _):
        stage_v[pl.ds(i * 16, 16)] = jnp.zeros((16,), jnp.float32)
        return 0
    lax.fori_loop(0, RPW // 16, _zero, 0)
    pltpu.sync_copy(stage_v, hist_sh.at[pl.ds(s * RPW, RPW)])

    def _ones(i, _):
        ones_v[pl.ds(i * 16, 16)] = jnp.ones((16,), jnp.float32)
        return 0
    lax.fori_loop(0, RCH // 16, _ones, 0)

    plsc.subcore_barrier()

    for j in range(NCH):
        pltpu.sync_copy(ones_v, hist_sh.at[idx_v.at[j]], add=True)

    plsc.subcore_barrier()

    pltpu.sync_copy(hist_sh.at[pl.ds(s * RPW, RPW)], stage_v)
    pltpu.sync_copy(stage_v, hist_ref.at[c, pl.ds(s * RPW, RPW)])


def _sc_gather_hist(codebook, indices):
    mesh = plsc.VectorSubcoreMesh(core_axis_name="c", subcore_axis_name="s")
    f = pl.kernel(
        _sc_body,
        out_type=[
            jax.ShapeDtypeStruct((N, D), jnp.float32),
            jax.ShapeDtypeStruct((NC, K), jnp.float32),
        ],
        mesh=mesh,
        scratch_types=[
            pltpu.VMEM((NCH, RCH), jnp.int32),
            pltpu.VMEM((NCH, RCH, D), jnp.float32),
            pltpu.VMEM((RPW,), jnp.float32),
            pltpu.VMEM((RCH,), jnp.float32),
            pltpu.VMEM_SHARED((K,), jnp.float32),
            pltpu.SemaphoreType.DMA,
        ],
        compiler_params=pltpu.CompilerParams(use_tc_tiling_on_sc=False),
    )
    return f(codebook, indices)


def _tail_body(z_ref, zh_ref, noise_ref, hist_ref, zq_ref, perp_ref):
    zb = z_ref[...]
    zh = zh_ref[...]
    nb = noise_ref[...]
    direction = zh - zb
    rv = nb + direction
    nrm = jnp.maximum(jnp.sqrt(jnp.sum(rv * rv, axis=1, keepdims=True)), 1e-12)
    err = jnp.sqrt(jnp.sum(direction * direction, axis=1, keepdims=True))
    zq_ref[...] = zb + err * (rv / nrm)

    counts = hist_ref[0:1, :] + hist_ref[1:2, :]
    probs = counts * (1.0 / N)
    ent = jnp.sum(probs * jnp.log(probs + 1e-10), keepdims=True)
    perp_ref[...] = jnp.exp(-ent)


def _tail(z, z_hard, noise, hist):
    zq, perp = pl.pallas_call(
        _tail_body,
        out_shape=[
            jax.ShapeDtypeStruct((N, D), jnp.float32),
            jax.ShapeDtypeStruct((1, 1), jnp.float32),
        ],
    )(z, z_hard, noise, hist)
    return zq, perp[0, 0]


def kernel(z, codebook, noise):
    zsq = jnp.sum(z ** 2, axis=1)[:, None]
    indices = _argmin_indices(z, zsq, (codebook * 2.0).T)
    z_hard, hist = _sc_gather_hist(codebook, indices)
    z_q, perplexity = _tail(z, z_hard, noise, hist)
    return (z_q, indices, perplexity)


# speedup vs baseline: 2.0113x; 1.0113x over previous
import jax
import jax.numpy as jnp
from jax import lax
from jax.experimental import pallas as pl
from jax.experimental.pallas import tpu as pltpu
from jax.experimental.pallas import tpu_sc as plsc

N = 16384
D = 32
K = 8192
BN = 512
BK = 4096

NC = 2
NS = 16
NW = NC * NS
RPW = N // NW
RCH = 128
NCH = RPW // RCH


RB = 64
LG = 128


def _argmin_body(z_ref, zsq_ref, ct2_ref, out_ref):
    zb = z_ref[...]
    ct2 = ct2_ref[...]
    csq = jnp.sum(ct2 * ct2, axis=0, keepdims=True) * 0.25
    zb16 = zb.astype(jnp.bfloat16)
    p2 = [jnp.dot(zb16, ct2[:, j * BK:(j + 1) * BK].astype(jnp.bfloat16),
                  preferred_element_type=jnp.float32)
          for j in range(K // BK)]
    zsq = zsq_ref[...]

    results = []
    for rb in range(BN // RB):
        rows = slice(rb * RB, (rb + 1) * RB)
        zsq_rb = zsq[rows, :]
        lane = jax.lax.broadcasted_iota(jnp.int32, (RB, LG), 1)
        bestv = jnp.full((RB, 1), jnp.inf, jnp.float32)
        besti = jnp.zeros((RB, 1), jnp.int32)
        for j in range(K // BK):
            acc_v = None
            for g in range(BK // LG):
                cols = slice(g * LG, (g + 1) * LG)
                u = zsq_rb + csq[:, j * BK + g * LG: j * BK + (g + 1) * LG]
                dg = u - p2[j][rows, cols]
                if acc_v is None:
                    acc_v = dg
                    acc_c = jnp.zeros((RB, LG), jnp.int32)
                else:
                    ch = dg < acc_v
                    acc_v = jnp.where(ch, dg, acc_v)
                    acc_c = jnp.where(ch, jnp.full((RB, LG), g, jnp.int32), acc_c)
            cmin = jnp.min(acc_v, axis=1, keepdims=True)
            kk = acc_c * LG + lane
            cand = jnp.min(jnp.where(acc_v == cmin, kk, BK),
                           axis=1, keepdims=True) + j * BK
            take = cmin < bestv
            bestv = jnp.where(take, cmin, bestv)
            besti = jnp.where(take, cand, besti)
            bestv = bestv.astype(jnp.bfloat16).astype(jnp.float32)
        results.append(besti)
    out_ref[...] = jnp.concatenate(results, axis=0)


def _argmin_indices(z, zsq, codebook_t2):
    out = pl.pallas_call(
        _argmin_body,
        grid=(N // BN,),
        in_specs=[
            pl.BlockSpec((BN, D), lambda i: (i, 0)),
            pl.BlockSpec((BN, 1), lambda i: (i, 0)),
            pl.BlockSpec((D, K), lambda i: (0, 0)),
        ],
        out_specs=pl.BlockSpec((BN, 1), lambda i: (i, 0)),
        out_shape=jax.ShapeDtypeStruct((N, 1), jnp.int32),
        compiler_params=pltpu.CompilerParams(allow_input_fusion=[False, True, True]),
    )(z, zsq, codebook_t2)
    return out[:, 0]


def _sc_body(cb_ref, idx_hbm, zhard_ref, hist_ref,
             idx_v, rows_v, stage_v, ones_v, hist_sh, sem):
    c = lax.axis_index("c")
    s = lax.axis_index("s")
    wid = s * NC + c
    base = wid * RPW

    for j in range(NCH):
        pltpu.sync_copy(idx_hbm.at[pl.ds(base + j * RCH, RCH)], idx_v.at[j])

    descs = [pltpu.async_copy(cb_ref.at[idx_v.at[j]], rows_v.at[j], sem)
             for j in range(NCH)]
    for dsc in descs:
        dsc.wait()
    for j in range(NCH):
        pltpu.sync_copy(rows_v.at[j], zhard_ref.at[pl.ds(base + j * RCH, RCH)])

    def _zero(i, _):
        stage_v[pl.ds(i * 16, 16)] = jnp.zeros((16,), jnp.float32)
        return 0
    lax.fori_loop(0, RPW // 16, _zero, 0)
    pltpu.sync_copy(stage_v, hist_sh.at[pl.ds(s * RPW, RPW)])

    def _ones(i, _):
        ones_v[pl.ds(i * 16, 16)] = jnp.ones((16,), jnp.float32)
        return 0
    lax.fori_loop(0, RCH // 16, _ones, 0)

    plsc.subcore_barrier()

    for j in range(NCH):
        pltpu.sync_copy(ones_v, hist_sh.at[idx_v.at[j]], add=True)

    plsc.subcore_barrier()

    pltpu.sync_copy(hist_sh.at[pl.ds(s * RPW, RPW)], stage_v)
    pltpu.sync_copy(stage_v, hist_ref.at[c, pl.ds(s * RPW, RPW)])


def _sc_gather_hist(codebook, indices):
    mesh = plsc.VectorSubcoreMesh(core_axis_name="c", subcore_axis_name="s")
    f = pl.kernel(
        _sc_body,
        out_type=[
            jax.ShapeDtypeStruct((N, D), jnp.float32),
            jax.ShapeDtypeStruct((NC, K), jnp.float32),
        ],
        mesh=mesh,
        scratch_types=[
            pltpu.VMEM((NCH, RCH), jnp.int32),
            pltpu.VMEM((NCH, RCH, D), jnp.float32),
            pltpu.VMEM((RPW,), jnp.float32),
            pltpu.VMEM((RCH,), jnp.float32),
            pltpu.VMEM_SHARED((K,), jnp.float32),
            pltpu.SemaphoreType.DMA,
        ],
        compiler_params=pltpu.CompilerParams(use_tc_tiling_on_sc=False),
    )
    return f(codebook, indices)


def _tail_body(z_ref, zh_ref, noise_ref, hist_ref, zq_ref, perp_ref):
    zb = z_ref[...]
    zh = zh_ref[...]
    nb = noise_ref[...]
    direction = zh - zb
    rv = nb + direction
    nrm = jnp.maximum(jnp.sqrt(jnp.sum(rv * rv, axis=1, keepdims=True)), 1e-12)
    err = jnp.sqrt(jnp.sum(direction * direction, axis=1, keepdims=True))
    zq_ref[...] = zb + err * (rv / nrm)

    counts = hist_ref[0:1, :] + hist_ref[1:2, :]
    probs = counts * (1.0 / N)
    ent = jnp.sum(probs * jnp.log(probs + 1e-10), keepdims=True)
    perp_ref[...] = jnp.exp(-ent)


def _tail(z, z_hard, noise, hist):
    zq, perp = pl.pallas_call(
        _tail_body,
        out_shape=[
            jax.ShapeDtypeStruct((N, D), jnp.float32),
            jax.ShapeDtypeStruct((1, 1), jnp.float32),
        ],
    )(z, z_hard, noise, hist)
    return zq, perp[0, 0]


def kernel(z, codebook, noise):
    zsq = jnp.sum(z ** 2, axis=1)[:, None]
    indices = _argmin_indices(z, zsq, (codebook * 2.0).T)
    z_hard, hist = _sc_gather_hist(codebook, indices)
    z_q, perplexity = _tail(z, z_hard, noise, hist)
    return (z_q, indices, perplexity)


# speedup vs baseline: 2.0744x; 1.0314x over previous
import jax
import jax.numpy as jnp
from jax import lax
from jax.experimental import pallas as pl
from jax.experimental.pallas import tpu as pltpu
from jax.experimental.pallas import tpu_sc as plsc

N = 16384
D = 32
K = 8192
BN = 1024
BK = 4096

NC = 2
NS = 16
NW = NC * NS
RPW = N // NW
RCH = 128
NCH = RPW // RCH


RB = 32
LG = 128


def _argmin_body(z_ref, zsq_ref, ct2_ref, out_ref):
    zb = z_ref[...]
    ct2 = ct2_ref[...]
    csq = jnp.sum(ct2 * ct2, axis=0, keepdims=True) * 0.25
    zb16 = zb.astype(jnp.bfloat16)
    p2 = [jnp.dot(zb16, ct2[:, j * BK:(j + 1) * BK].astype(jnp.bfloat16),
                  preferred_element_type=jnp.float32)
          for j in range(K // BK)]
    zsq = zsq_ref[...]

    results = []
    for rb in range(BN // RB):
        rows = slice(rb * RB, (rb + 1) * RB)
        zsq_rb = zsq[rows, :]
        lane = jax.lax.broadcasted_iota(jnp.int32, (RB, LG), 1)
        bestv = jnp.full((RB, 1), jnp.inf, jnp.float32)
        besti = jnp.zeros((RB, 1), jnp.int32)
        for j in range(K // BK):
            acc_v = None
            for g in range(BK // LG):
                cols = slice(g * LG, (g + 1) * LG)
                u = zsq_rb + csq[:, j * BK + g * LG: j * BK + (g + 1) * LG]
                dg = u - p2[j][rows, cols]
                if acc_v is None:
                    acc_v = dg
                    acc_c = jnp.zeros((RB, LG), jnp.int32)
                else:
                    ch = dg < acc_v
                    acc_v = jnp.where(ch, dg, acc_v)
                    acc_c = jnp.where(ch, jnp.full((RB, LG), g, jnp.int32), acc_c)
            cmin = jnp.min(acc_v, axis=1, keepdims=True)
            kk = acc_c * LG + lane
            cand = jnp.min(jnp.where(acc_v == cmin, kk, BK),
                           axis=1, keepdims=True) + j * BK
            take = cmin < bestv
            bestv = jnp.where(take, cmin, bestv)
            besti = jnp.where(take, cand, besti)
            bestv = bestv.astype(jnp.bfloat16).astype(jnp.float32)
        results.append(besti)
    out_ref[...] = jnp.concatenate(results, axis=0)


def _argmin_indices(z, zsq, codebook_t2):
    out = pl.pallas_call(
        _argmin_body,
        grid=(N // BN,),
        in_specs=[
            pl.BlockSpec((BN, D), lambda i: (i, 0)),
            pl.BlockSpec((BN, 1), lambda i: (i, 0)),
            pl.BlockSpec((D, K), lambda i: (0, 0)),
        ],
        out_specs=pl.BlockSpec((BN, 1), lambda i: (i, 0)),
        out_shape=jax.ShapeDtypeStruct((N, 1), jnp.int32),
        compiler_params=pltpu.CompilerParams(allow_input_fusion=[False, True, True]),
    )(z, zsq, codebook_t2)
    return out[:, 0]


def _sc_body(cb_ref, idx_hbm, zhard_ref, hist_ref,
             idx_v, rows_v, stage_v, ones_v, hist_sh, sem):
    c = lax.axis_index("c")
    s = lax.axis_index("s")
    wid = s * NC + c
    base = wid * RPW

    for j in range(NCH):
        pltpu.sync_copy(idx_hbm.at[pl.ds(base + j * RCH, RCH)], idx_v.at[j])

    descs = [pltpu.async_copy(cb_ref.at[idx_v.at[j]], rows_v.at[j], sem)
             for j in range(NCH)]
    for dsc in descs:
        dsc.wait()
    for j in range(NCH):
        pltpu.sync_copy(rows_v.at[j], zhard_ref.at[pl.ds(base + j * RCH, RCH)])

    def _zero(i, _):
        stage_v[pl.ds(i * 16, 16)] = jnp.zeros((16,), jnp.float32)
        return 0
    lax.fori_loop(0, RPW // 16, _zero, 0)
    pltpu.sync_copy(stage_v, hist_sh.at[pl.ds(s * RPW, RPW)])

    def _ones(i, _):
        ones_v[pl.ds(i * 16, 16)] = jnp.ones((16,), jnp.float32)
        return 0
    lax.fori_loop(0, RCH // 16, _ones, 0)

    plsc.subcore_barrier()

    for j in range(NCH):
        pltpu.sync_copy(ones_v, hist_sh.at[idx_v.at[j]], add=True)

    plsc.subcore_barrier()

    pltpu.sync_copy(hist_sh.at[pl.ds(s * RPW, RPW)], stage_v)
    pltpu.sync_copy(stage_v, hist_ref.at[c, pl.ds(s * RPW, RPW)])


def _sc_gather_hist(codebook, indices):
    mesh = plsc.VectorSubcoreMesh(core_axis_name="c", subcore_axis_name="s")
    f = pl.kernel(
        _sc_body,
        out_type=[
            jax.ShapeDtypeStruct((N, D), jnp.float32),
            jax.ShapeDtypeStruct((NC, K), jnp.float32),
        ],
        mesh=mesh,
        scratch_types=[
            pltpu.VMEM((NCH, RCH), jnp.int32),
            pltpu.VMEM((NCH, RCH, D), jnp.float32),
            pltpu.VMEM((RPW,), jnp.float32),
            pltpu.VMEM((RCH,), jnp.float32),
            pltpu.VMEM_SHARED((K,), jnp.float32),
            pltpu.SemaphoreType.DMA,
        ],
        compiler_params=pltpu.CompilerParams(use_tc_tiling_on_sc=False),
    )
    return f(codebook, indices)


def _tail_body(z_ref, zh_ref, noise_ref, hist_ref, zq_ref, perp_ref):
    zb = z_ref[...]
    zh = zh_ref[...]
    nb = noise_ref[...]
    direction = zh - zb
    rv = nb + direction
    nrm = jnp.maximum(jnp.sqrt(jnp.sum(rv * rv, axis=1, keepdims=True)), 1e-12)
    err = jnp.sqrt(jnp.sum(direction * direction, axis=1, keepdims=True))
    zq_ref[...] = zb + err * (rv / nrm)

    counts = hist_ref[0:1, :] + hist_ref[1:2, :]
    probs = counts * (1.0 / N)
    ent = jnp.sum(probs * jnp.log(probs + 1e-10), keepdims=True)
    perp_ref[...] = jnp.exp(-ent)


def _tail(z, z_hard, noise, hist):
    zq, perp = pl.pallas_call(
        _tail_body,
        out_shape=[
            jax.ShapeDtypeStruct((N, D), jnp.float32),
            jax.ShapeDtypeStruct((1, 1), jnp.float32),
        ],
    )(z, z_hard, noise, hist)
    return zq, perp[0, 0]


def kernel(z, codebook, noise):
    zsq = jnp.sum(z ** 2, axis=1)[:, None]
    indices = _argmin_indices(z, zsq, (codebook * 2.0).T)
    z_hard, hist = _sc_gather_hist(codebook, indices)
    z_q, perplexity = _tail(z, z_hard, noise, hist)
    return (z_q, indices, perplexity)
